# baseline (speedup 1.0000x reference)
import functools

import jax
import jax.numpy as jnp
from jax import lax
from jax.experimental import pallas as pl
from jax.experimental.pallas import tpu as pltpu
from jax.experimental.pallas import tpu_sc as plsc

MINI_BATCH = 16384
STATE_DIM = 64
NC = 2
NS = 16
NW = NC * NS
B_PER_W = MINI_BATCH // NW
CHUNK = 128
NCHUNK = B_PER_W // CHUNK


def _sample_body(state_hbm, next_hbm, rew_hbm, msk_hbm, act_hbm, idx_hbm,
                 out_state, out_act, out_rew, out_next, out_msk,
                 idx_v, st_v, nx_v, rew_v, msk_v, act_v, sem):
    wid = lax.axis_index("s") * NC + lax.axis_index("c")
    base = wid * B_PER_W

    for c in range(NCHUNK):
        pltpu.sync_copy(idx_hbm.at[pl.ds(base + c * CHUNK, CHUNK)],
                        idx_v.at[c])

    copies = []
    for c in range(NCHUNK):
        ids = idx_v.at[c]
        copies.append(pltpu.async_copy(
            state_hbm.at[ids], st_v.at[pl.ds(c * CHUNK, CHUNK)], sem))
        copies.append(pltpu.async_copy(
            next_hbm.at[ids], nx_v.at[pl.ds(c * CHUNK, CHUNK)], sem))
        copies.append(pltpu.async_copy(rew_hbm.at[ids], rew_v.at[c], sem))
        copies.append(pltpu.async_copy(msk_hbm.at[ids], msk_v.at[c], sem))
        copies.append(pltpu.async_copy(act_hbm.at[ids], act_v.at[c], sem))
    for cp in copies:
        cp.wait()

    pltpu.sync_copy(st_v, out_state.at[pl.ds(base, B_PER_W)])
    pltpu.sync_copy(nx_v, out_next.at[pl.ds(base, B_PER_W)])
    for c in range(NCHUNK):
        off = base + c * CHUNK
        pltpu.sync_copy(rew_v.at[c], out_rew.at[pl.ds(off, CHUNK)])
        pltpu.sync_copy(msk_v.at[c], out_msk.at[pl.ds(off, CHUNK)])
        pltpu.sync_copy(act_v.at[c], out_act.at[pl.ds(off, CHUNK)])


@jax.jit
def kernel(state, next_state, reward, masks, action, idx):
    idx = idx.astype(jnp.int32)
    act_dtype = action.dtype
    mesh = plsc.VectorSubcoreMesh(core_axis_name="c", subcore_axis_name="s")
    run = pl.kernel(
        _sample_body,
        mesh=mesh,
        compiler_params=pltpu.CompilerParams(use_tc_tiling_on_sc=False),
        out_type=[
            jax.ShapeDtypeStruct((MINI_BATCH, STATE_DIM), jnp.float32),
            jax.ShapeDtypeStruct((MINI_BATCH,), act_dtype),
            jax.ShapeDtypeStruct((MINI_BATCH,), jnp.float32),
            jax.ShapeDtypeStruct((MINI_BATCH, STATE_DIM), jnp.float32),
            jax.ShapeDtypeStruct((MINI_BATCH,), jnp.float32),
        ],
        scratch_types=[
            pltpu.VMEM((NCHUNK, CHUNK), jnp.int32),
            pltpu.VMEM((B_PER_W, STATE_DIM), jnp.float32),
            pltpu.VMEM((B_PER_W, STATE_DIM), jnp.float32),
            pltpu.VMEM((NCHUNK, CHUNK), jnp.float32),
            pltpu.VMEM((NCHUNK, CHUNK), jnp.float32),
            pltpu.VMEM((NCHUNK, CHUNK), act_dtype),
            pltpu.SemaphoreType.DMA,
        ],
    )
    out_state, out_act, out_rew, out_next, out_msk = run(
        state, next_state, reward, masks, action, idx)
    return (out_state, out_act, out_rew, out_next, out_msk)
